# baseline (speedup 1.0000x reference)
import functools

import jax
import jax.numpy as jnp
from jax import lax
from jax.experimental import pallas as pl
from jax.experimental.pallas import tpu as pltpu
from jax.experimental.pallas import tpu_sc as plsc

H = 128
NODES = 10000
EDGES = 320000
NCORES = 2
NSUB = 16
NW = NCORES * NSUB
NPW = 313
NPAD = NW * NPW
CHUNK = 8000
NCHUNK = EDGES // CHUNK
NEG = -1e30


def _segmax_body(dst_h, src_h, ea_h, xv_h, uw_h, m_h,
                 dstb, srcb, eab, xb, uwb, accb, std, sem):
    cid = lax.axis_index("c")
    sid = lax.axis_index("s")
    wid = sid * NCORES + cid
    lo = wid * NPW

    pltpu.sync_copy(xv_h, xb)
    pltpu.sync_copy(uw_h, uwb)

    u2v = [uwb[pl.ds(hh * 16, 16)] for hh in range(8)]
    w3v = [uwb[pl.ds(H + hh * 16, 16)] for hh in range(8)]

    neg = jnp.full((16,), NEG, jnp.float32)

    @plsc.parallel_loop(0, (NPW + 1) * H // 16, unroll=8)
    def init_blk(i):
        accb[pl.ds(i * 16, 16)] = neg

    dstb[pl.ds(2 * CHUNK, 16)] = jnp.full((16,), lo + NPW, jnp.int32)
    srcb[pl.ds(2 * CHUNK, 16)] = jnp.zeros((16,), jnp.int32)
    eab[pl.ds(2 * CHUNK, 16)] = jnp.zeros((16,), jnp.float32)
    dummy_idx = jnp.full((16,), 2 * CHUNK, jnp.int32)
    iota16 = lax.iota(jnp.int32, 16)

    def issue(g, par):
        hb = g * CHUNK
        sl = par * CHUNK
        pltpu.make_async_copy(dst_h.at[pl.ds(hb, CHUNK)],
                              dstb.at[pl.ds(sl, CHUNK)], sem.at[par]).start()
        pltpu.make_async_copy(src_h.at[pl.ds(hb, CHUNK)],
                              srcb.at[pl.ds(sl, CHUNK)], sem.at[par]).start()
        pltpu.make_async_copy(ea_h.at[pl.ds(hb, CHUNK)],
                              eab.at[pl.ds(sl, CHUNK)], sem.at[par]).start()

    def drain_dma(g, par):
        hb = g * CHUNK
        sl = par * CHUNK
        pltpu.make_async_copy(dst_h.at[pl.ds(hb, CHUNK)],
                              dstb.at[pl.ds(sl, CHUNK)], sem.at[par]).wait()
        pltpu.make_async_copy(src_h.at[pl.ds(hb, CHUNK)],
                              srcb.at[pl.ds(sl, CHUNK)], sem.at[par]).wait()
        pltpu.make_async_copy(ea_h.at[pl.ds(hb, CHUNK)],
                              eab.at[pl.ds(sl, CHUNK)], sem.at[par]).wait()

    issue(0, 0)

    def chunk_body(g, _):
        par = g % 2

        @pl.when(g + 1 < NCHUNK)
        def _():
            issue(g + 1, 1 - par)

        drain_dma(g, par)
        sl = par * CHUNK

        @plsc.parallel_loop(0, CHUNK // 16, carry=jnp.int32(0), unroll=8)
        def vec_body(i, ptr):
            d16 = dstb[pl.ds(sl + i * 16, 16)]
            hit = (d16 >= lo) & (d16 < lo + NPW)
            idx16 = iota16 + (sl + i * 16)
            plsc.store_compressed(std.at[pl.ds(ptr, 16)], idx16, mask=hit)
            nhit = plsc.all_reduce_population_count(hit)
            return ptr + nhit[0]

        ptr = vec_body

        std[pl.ds(ptr, 16)] = dummy_idx
        ngrp = (ptr + 15) // 16

        def drain(gi, _):
            idxg = std[pl.ds(gi * 16, 16)]
            d16 = plsc.load_gather(dstb, [idxg])
            s16 = plsc.load_gather(srcb, [idxg])
            c16 = plsc.load_gather(eab, [idxg])
            b16 = plsc.load_gather(xb, [s16])
            off16 = (d16 - lo) * H
            for j in range(16):
                off = off16[j]
                b = b16[j]
                c = c16[j]
                for hh in range(8):
                    cur = accb[pl.ds(off + hh * 16, 16)]
                    val = b * u2v[hh] + c * w3v[hh]
                    accb[pl.ds(off + hh * 16, 16)] = jnp.maximum(cur, val)
            return 0

        lax.fori_loop(0, ngrp, drain, 0)
        return 0

    lax.fori_loop(0, NCHUNK, chunk_body, 0)

    pltpu.sync_copy(accb.at[pl.ds(0, NPW * H)], m_h.at[pl.ds(lo * H, NPW * H)])


_segmax = functools.partial(
    pl.kernel,
    out_type=jax.ShapeDtypeStruct((NPAD * H,), jnp.float32),
    mesh=plsc.VectorSubcoreMesh(core_axis_name="c", subcore_axis_name="s"),
    compiler_params=pltpu.CompilerParams(needs_layout_passes=False),
    scratch_types=[
        pltpu.VMEM((2 * CHUNK + 16,), jnp.int32),
        pltpu.VMEM((2 * CHUNK + 16,), jnp.int32),
        pltpu.VMEM((2 * CHUNK + 16,), jnp.float32),
        pltpu.VMEM((NPAD,), jnp.float32),
        pltpu.VMEM((2 * H,), jnp.float32),
        pltpu.VMEM(((NPW + 1) * H,), jnp.float32),
        pltpu.VMEM((CHUNK + 16,), jnp.int32),
        pltpu.SemaphoreType.DMA((2,)),
    ],
)(_segmax_body)


def _tail_body(x_ref, m_ref, u1_ref, c0_ref, p_ref, q_ref, bu_ref,
               wu2_ref, wd_ref, bd_ref, wt_ref, bt_ref, out_ref, term_ref):
    xv = x_ref[:, :]
    aggr = jnp.maximum(xv * u1_ref[:, :] + c0_ref[:, :] + m_ref[:, :], 0.0)
    h2 = jnp.maximum(
        xv * p_ref[:, :] + q_ref[:, :] + bu_ref[:, :]
        + jnp.dot(aggr, wu2_ref[:, :], preferred_element_type=jnp.float32),
        0.0)
    logits = jnp.sum(h2 * wd_ref[:, :], axis=1, keepdims=True) + bd_ref[0, 0]
    out_ref[:, :] = jax.nn.sigmoid(logits)
    rid = lax.broadcasted_iota(jnp.int32, (NPAD, 1), 0)
    h2m = jnp.where(rid < NODES, h2, 0.0)
    mean = jnp.sum(h2m, axis=0, keepdims=True) * (1.0 / NODES)
    t = jnp.sum(mean * wt_ref[:, :], axis=1, keepdims=True) + bt_ref[0, 0]
    term_ref[:, :] = jax.nn.sigmoid(t)


_tail = pl.pallas_call(
    _tail_body,
    out_shape=[
        jax.ShapeDtypeStruct((NPAD, 1), jnp.float32),
        jax.ShapeDtypeStruct((1, 1), jnp.float32),
    ],
)


def kernel(x, edge_index, edge_attr, W_enc, b_enc, W_msg, b_msg,
           W_upd, b_upd, W_dec, b_dec, W_t, b_t):
    xv = x[:, 0]
    we = W_enc[0]
    Wm1, Wm2, w3 = W_msg[:H], W_msg[H:2 * H], W_msg[2 * H]
    u1 = we @ Wm1
    u2 = we @ Wm2
    c0 = b_enc @ Wm1 + b_enc @ Wm2 + b_msg
    Wu1, Wu2 = W_upd[:H], W_upd[H:]
    p = we @ Wu1
    q = b_enc @ Wu1

    xpad = jnp.pad(xv, (0, NPAD - NODES))
    src = edge_index[0]
    dst = edge_index[1]
    eav = edge_attr[:, 0]
    uw = jnp.concatenate([u2, w3])

    M = _segmax(dst, src, eav, xpad, uw).reshape(NPAD, H)

    out_pad, term = _tail(
        xpad[:, None], M, u1[None], c0[None], p[None], q[None],
        b_upd[None], Wu2, W_dec[:, 0][None], b_dec[None],
        W_t[:, 0][None], b_t[None])
    return (out_pad[:NODES], term)


# speedup vs baseline: 1.1489x; 1.1489x over previous
import functools

import jax
import jax.numpy as jnp
from jax import lax
from jax.experimental import pallas as pl
from jax.experimental.pallas import tpu as pltpu
from jax.experimental.pallas import tpu_sc as plsc

H = 128
NODES = 10000
EDGES = 320000
NCORES = 2
NSUB = 16
NW = NCORES * NSUB
NPW = 313
NPAD = NW * NPW
CHUNK = 6400
NCHUNK = EDGES // CHUNK
NEG = -1e30


def _segmax_body(dst_h, src_h, ea_h, xv_h, uw_h, m_h,
                 dstb, srcb, eab, xb, uwb, accb, std, sem):
    cid = lax.axis_index("c")
    sid = lax.axis_index("s")
    wid = sid * NCORES + cid
    lo = wid * NPW

    pltpu.sync_copy(xv_h, xb)
    pltpu.sync_copy(uw_h, uwb)

    u2v = [uwb[pl.ds(hh * 16, 16)] for hh in range(8)]
    w3v = [uwb[pl.ds(H + hh * 16, 16)] for hh in range(8)]

    neg = jnp.full((16,), NEG, jnp.float32)

    @plsc.parallel_loop(0, (NPW + 1) * H // 16, unroll=8)
    def init_blk(i):
        accb[pl.ds(i * 16, 16)] = neg

    dstb[pl.ds(2 * CHUNK, 16)] = jnp.full((16,), lo + NPW, jnp.int32)
    srcb[pl.ds(2 * CHUNK, 16)] = jnp.zeros((16,), jnp.int32)
    eab[pl.ds(2 * CHUNK, 16)] = jnp.zeros((16,), jnp.float32)
    dummy_idx = jnp.full((16,), 2 * CHUNK, jnp.int32)
    iota16 = lax.iota(jnp.int32, 16)

    def issue(g, par):
        hb = g * CHUNK
        sl = par * CHUNK
        pltpu.make_async_copy(dst_h.at[pl.ds(hb, CHUNK)],
                              dstb.at[pl.ds(sl, CHUNK)], sem.at[par]).start()
        pltpu.make_async_copy(src_h.at[pl.ds(hb, CHUNK)],
                              srcb.at[pl.ds(sl, CHUNK)], sem.at[par]).start()
        pltpu.make_async_copy(ea_h.at[pl.ds(hb, CHUNK)],
                              eab.at[pl.ds(sl, CHUNK)], sem.at[par]).start()

    def drain_dma(g, par):
        hb = g * CHUNK
        sl = par * CHUNK
        pltpu.make_async_copy(dst_h.at[pl.ds(hb, CHUNK)],
                              dstb.at[pl.ds(sl, CHUNK)], sem.at[par]).wait()
        pltpu.make_async_copy(src_h.at[pl.ds(hb, CHUNK)],
                              srcb.at[pl.ds(sl, CHUNK)], sem.at[par]).wait()
        pltpu.make_async_copy(ea_h.at[pl.ds(hb, CHUNK)],
                              eab.at[pl.ds(sl, CHUNK)], sem.at[par]).wait()

    issue(0, 0)

    def chunk_body(g, _):
        par = g % 2

        @pl.when(g + 1 < NCHUNK)
        def _():
            issue(g + 1, 1 - par)

        drain_dma(g, par)
        sl = par * CHUNK

        @plsc.parallel_loop(0, CHUNK // 16, carry=jnp.int32(0), unroll=4)
        def vec_body(i, ptr):
            d16 = dstb[pl.ds(sl + i * 16, 16)]
            hit = (d16 >= lo) & (d16 < lo + NPW)
            idx16 = iota16 + (sl + i * 16)
            plsc.store_compressed(std.at[pl.ds(ptr, 16)], idx16, mask=hit)
            nhit = plsc.all_reduce_population_count(hit)
            return ptr + nhit[0]

        ptr = vec_body

        std[pl.ds(ptr, 16)] = dummy_idx
        ngrp = (ptr + 15) // 16

        def drain(gi, _):
            idxg = std[pl.ds(gi * 16, 16)]
            d16 = plsc.load_gather(dstb, [idxg])
            s16 = plsc.load_gather(srcb, [idxg])
            c16 = plsc.load_gather(eab, [idxg])
            b16 = plsc.load_gather(xb, [s16])
            off16 = (d16 - lo) * H
            for j in range(16):
                off = off16[j]
                b = b16[j]
                c = c16[j]
                for hh in range(8):
                    cur = accb[pl.ds(off + hh * 16, 16)]
                    val = b * u2v[hh] + c * w3v[hh]
                    accb[pl.ds(off + hh * 16, 16)] = jnp.maximum(cur, val)
            return 0

        lax.fori_loop(0, ngrp, drain, 0)
        return 0

    lax.fori_loop(0, NCHUNK, chunk_body, 0)

    pltpu.sync_copy(accb.at[pl.ds(0, NPW * H)], m_h.at[pl.ds(lo * H, NPW * H)])


_segmax = functools.partial(
    pl.kernel,
    out_type=jax.ShapeDtypeStruct((NPAD * H,), jnp.float32),
    mesh=plsc.VectorSubcoreMesh(core_axis_name="c", subcore_axis_name="s"),
    compiler_params=pltpu.CompilerParams(needs_layout_passes=False),
    scratch_types=[
        pltpu.VMEM((2 * CHUNK + 16,), jnp.int32),
        pltpu.VMEM((2 * CHUNK + 16,), jnp.int32),
        pltpu.VMEM((2 * CHUNK + 16,), jnp.float32),
        pltpu.VMEM((NPAD,), jnp.float32),
        pltpu.VMEM((2 * H,), jnp.float32),
        pltpu.VMEM(((NPW + 1) * H,), jnp.float32),
        pltpu.VMEM((CHUNK + 16,), jnp.int32),
        pltpu.SemaphoreType.DMA((2,)),
    ],
)(_segmax_body)


def _tail_body(x_ref, m_ref, u1_ref, c0_ref, p_ref, q_ref, bu_ref,
               wu2_ref, wd_ref, bd_ref, wt_ref, bt_ref, out_ref, term_ref):
    xv = x_ref[:, :]
    aggr = jnp.maximum(xv * u1_ref[:, :] + c0_ref[:, :] + m_ref[:, :], 0.0)
    h2 = jnp.maximum(
        xv * p_ref[:, :] + q_ref[:, :] + bu_ref[:, :]
        + jnp.dot(aggr, wu2_ref[:, :], preferred_element_type=jnp.float32),
        0.0)
    logits = jnp.sum(h2 * wd_ref[:, :], axis=1, keepdims=True) + bd_ref[0, 0]
    out_ref[:, :] = jax.nn.sigmoid(logits)
    rid = lax.broadcasted_iota(jnp.int32, (NPAD, 1), 0)
    h2m = jnp.where(rid < NODES, h2, 0.0)
    mean = jnp.sum(h2m, axis=0, keepdims=True) * (1.0 / NODES)
    t = jnp.sum(mean * wt_ref[:, :], axis=1, keepdims=True) + bt_ref[0, 0]
    term_ref[:, :] = jax.nn.sigmoid(t)


_tail = pl.pallas_call(
    _tail_body,
    out_shape=[
        jax.ShapeDtypeStruct((NPAD, 1), jnp.float32),
        jax.ShapeDtypeStruct((1, 1), jnp.float32),
    ],
)


def kernel(x, edge_index, edge_attr, W_enc, b_enc, W_msg, b_msg,
           W_upd, b_upd, W_dec, b_dec, W_t, b_t):
    xv = x[:, 0]
    we = W_enc[0]
    Wm1, Wm2, w3 = W_msg[:H], W_msg[H:2 * H], W_msg[2 * H]
    u1 = we @ Wm1
    u2 = we @ Wm2
    c0 = b_enc @ Wm1 + b_enc @ Wm2 + b_msg
    Wu1, Wu2 = W_upd[:H], W_upd[H:]
    p = we @ Wu1
    q = b_enc @ Wu1

    xpad = jnp.pad(xv, (0, NPAD - NODES))
    src = edge_index[0]
    dst = edge_index[1]
    eav = edge_attr[:, 0]
    uw = jnp.concatenate([u2, w3])

    M = _segmax(dst, src, eav, xpad, uw).reshape(NPAD, H)

    out_pad, term = _tail(
        xpad[:, None], M, u1[None], c0[None], p[None], q[None],
        b_upd[None], Wu2, W_dec[:, 0][None], b_dec[None],
        W_t[:, 0][None], b_t[None])
    return (out_pad[:NODES], term)
